# baseline (speedup 1.0000x reference)
import jax
import jax.numpy as jnp
from jax.experimental import pallas as pl
from jax.experimental.pallas import tpu as pltpu


def _body(lg_ref, bx_ref, lab_ref, box_ref, cnt_ref):
    i = pl.program_id(0)
    x = lg_ref[...]
    br, c = x.shape
    m = jnp.max(x, axis=-1)
    eq = x == m[:, None]
    cidx = jax.lax.broadcasted_iota(jnp.int32, x.shape, 1)
    a = jnp.min(jnp.where(eq, cidx, c), axis=-1)
    valid = m > 0.0
    lab_ref[...] = jnp.where(valid, a, -1)
    box_ref[...] = jnp.where(valid[:, None], bx_ref[...], 0.0)

    @pl.when(i == 0)
    def _():
        cnt_ref[0, 0] = 0.0

    cnt_ref[0, 0] += jnp.sum(valid.astype(jnp.float32))


def kernel(pred_logits, pred_boxes):
    B, Q, C = pred_logits.shape
    R = B * Q
    lg = pred_logits.reshape(R, C)
    bx = pred_boxes.reshape(R, 4)

    BR = 4096
    labels, boxes, cnt = pl.pallas_call(
        _body,
        grid=(R // BR,),
        in_specs=[
            pl.BlockSpec((BR, C), lambda i: (i, 0)),
            pl.BlockSpec((BR, 4), lambda i: (i, 0)),
        ],
        out_specs=[
            pl.BlockSpec((BR,), lambda i: (i,)),
            pl.BlockSpec((BR, 4), lambda i: (i, 0)),
            pl.BlockSpec((1, 1), lambda i: (0, 0), memory_space=pltpu.SMEM),
        ],
        out_shape=[
            jax.ShapeDtypeStruct((R,), jnp.int32),
            jax.ShapeDtypeStruct((R, 4), jnp.float32),
            jax.ShapeDtypeStruct((1, 1), jnp.float32),
        ],
    )(lg, bx)
    num_boxes = jnp.maximum(cnt[0, 0], 1.0)
    return labels.reshape(B, Q), boxes.reshape(B, Q, 4), num_boxes
